# baseline (device time: 28692 ns/iter reference)
import jax
import jax.numpy as jnp
from jax import lax
from jax.experimental import pallas as pl
from jax.experimental.pallas import tpu as pltpu

N_GLOBAL = 2048
EPS = 1e-5
K = 8


def kernel(x, gamma, beta):
    m, n = x.shape
    bm = m // K
    half = m // 2
    gamma2 = gamma.reshape(1, n)
    beta2 = beta.reshape(1, n)

    def stats_body(
        x_hbm, stats_ref, xv, send_buf, recv,
        in_sems, send_sems, recv_sems,
    ):
        my_x = lax.axis_index("x")
        my_y = lax.axis_index("y")
        peer = (my_x, 1 - my_y)

        barrier = pltpu.get_barrier_semaphore()
        pl.semaphore_signal(
            barrier, inc=1, device_id=peer, device_id_type=pl.DeviceIdType.MESH
        )
        pl.semaphore_wait(barrier, 1)

        copies_in = []
        for k in range(K):
            cp = pltpu.make_async_copy(
                x_hbm.at[pl.ds(k * bm, bm), :],
                xv.at[pl.ds(k * bm, bm), :],
                in_sems.at[k],
            )
            cp.start()
            copies_in.append(cp)

        rdmas = []

        def exchange_half(h):
            sl = pl.ds(h * half, half)
            send_buf[:, sl] = jnp.transpose(stats_ref[sl, :], (1, 0))
            rdma = pltpu.make_async_remote_copy(
                src_ref=send_buf.at[:, sl],
                dst_ref=recv.at[:, sl],
                send_sem=send_sems.at[h],
                recv_sem=recv_sems.at[h],
                device_id=peer,
                device_id_type=pl.DeviceIdType.MESH,
            )
            rdma.start()
            rdmas.append(rdma)

        for k in range(K):
            copies_in[k].wait()
            xb = xv[pl.ds(k * bm, bm), :]
            stats_ref[pl.ds(k * bm, bm), 0:1] = jnp.sum(xb, axis=1, keepdims=True)
            stats_ref[pl.ds(k * bm, bm), 1:2] = jnp.sum(
                xb * xb, axis=1, keepdims=True
            )
            if k == K // 2 - 1:
                exchange_half(0)
        exchange_half(1)
        for rdma in rdmas:
            rdma.wait()

        rt = jnp.transpose(recv[...], (1, 0))
        tot1 = stats_ref[:, 0:1] + rt[:, 0:1]
        tot2 = stats_ref[:, 1:2] + rt[:, 1:2]
        mean = tot1 / N_GLOBAL
        var = tot2 / N_GLOBAL - mean * mean
        stats_ref[:, 0:1] = mean
        stats_ref[:, 1:2] = lax.rsqrt(var + EPS)

    stats = pl.pallas_call(
        stats_body,
        out_shape=jax.ShapeDtypeStruct((m, 2), jnp.float32),
        in_specs=[pl.BlockSpec(memory_space=pl.ANY)],
        out_specs=pl.BlockSpec(memory_space=pltpu.VMEM),
        scratch_shapes=[
            pltpu.VMEM((m, n), jnp.float32),
            pltpu.VMEM((2, m), jnp.float32),
            pltpu.VMEM((2, m), jnp.float32),
            pltpu.SemaphoreType.DMA((K,)),
            pltpu.SemaphoreType.DMA((2,)),
            pltpu.SemaphoreType.DMA((2,)),
        ],
        compiler_params=pltpu.CompilerParams(
            collective_id=0, vmem_limit_bytes=64 * 1024 * 1024
        ),
    )(x)

    def norm_body(x_hbm, stats_ref, g_ref, b_ref, o_ref, xv2, in_sems):
        k = pl.program_id(0)

        def copy_block(i, slot):
            return pltpu.make_async_copy(
                x_hbm.at[pl.ds(i * bm, bm), :], xv2.at[slot], in_sems.at[slot]
            )

        @pl.when(k == 0)
        def _():
            copy_block(0, 0).start()

        @pl.when(k < K - 1)
        def _():
            copy_block(k + 1, (k + 1) % 2).start()

        copy_block(k, k % 2).wait()
        sl = pl.ds(k * bm, bm)
        mu = stats_ref[sl, 0:1]
        rstd = stats_ref[sl, 1:2]
        o_ref[...] = (
            (xv2[k % 2] - mu) * rstd * g_ref[...] + b_ref[...]
        ).astype(jnp.bfloat16)

    return pl.pallas_call(
        norm_body,
        grid=(K,),
        out_shape=jax.ShapeDtypeStruct((m, n), jnp.bfloat16),
        in_specs=[
            pl.BlockSpec(memory_space=pl.ANY),
            pl.BlockSpec(memory_space=pltpu.VMEM),
            pl.BlockSpec(memory_space=pltpu.VMEM),
            pl.BlockSpec(memory_space=pltpu.VMEM),
        ],
        out_specs=pl.BlockSpec((bm, n), lambda k: (k, 0)),
        scratch_shapes=[
            pltpu.VMEM((2, bm, n), jnp.float32),
            pltpu.SemaphoreType.DMA((2,)),
        ],
    )(x, stats, gamma2, beta2)


# device time: 27027 ns/iter; 1.0616x vs baseline; 1.0616x over previous
import jax
import jax.numpy as jnp
from jax import lax
from jax.experimental import pallas as pl
from jax.experimental.pallas import tpu as pltpu

N_GLOBAL = 2048
EPS = 1e-5
K = 8


def kernel(x, gamma, beta):
    m, n = x.shape
    bm = m // K
    half = m // 2
    gamma2 = gamma.reshape(1, n)
    beta2 = beta.reshape(1, n)

    def body(
        x_hbm, g_ref, b_ref, o_ref,
        xv, stats, send_buf, recv,
        in_sems, send_sems, recv_sems,
    ):
        j = pl.program_id(0)

        @pl.when(j == 0)
        def stats_phase():
            my_x = lax.axis_index("x")
            my_y = lax.axis_index("y")
            peer = (my_x, 1 - my_y)

            barrier = pltpu.get_barrier_semaphore()
            pl.semaphore_signal(
                barrier, inc=1, device_id=peer,
                device_id_type=pl.DeviceIdType.MESH,
            )
            pl.semaphore_wait(barrier, 1)

            copies_in = []
            for k in range(K):
                cp = pltpu.make_async_copy(
                    x_hbm.at[pl.ds(k * bm, bm), :],
                    xv.at[pl.ds(k * bm, bm), :],
                    in_sems.at[k],
                )
                cp.start()
                copies_in.append(cp)

            rdmas = []

            def exchange_half(h):
                sl = pl.ds(h * half, half)
                send_buf[:, sl] = jnp.transpose(stats[sl, :], (1, 0))
                rdma = pltpu.make_async_remote_copy(
                    src_ref=send_buf.at[:, sl],
                    dst_ref=recv.at[:, sl],
                    send_sem=send_sems.at[h],
                    recv_sem=recv_sems.at[h],
                    device_id=peer,
                    device_id_type=pl.DeviceIdType.MESH,
                )
                rdma.start()
                rdmas.append(rdma)

            for k in range(K):
                copies_in[k].wait()
                xb = xv[pl.ds(k * bm, bm), :]
                stats[pl.ds(k * bm, bm), 0:1] = jnp.sum(xb, axis=1, keepdims=True)
                stats[pl.ds(k * bm, bm), 1:2] = jnp.sum(
                    xb * xb, axis=1, keepdims=True
                )
                if k == K // 2 - 1:
                    exchange_half(0)
            exchange_half(1)
            for rdma in rdmas:
                rdma.wait()

            rt = jnp.transpose(recv[...], (1, 0))
            tot1 = stats[:, 0:1] + rt[:, 0:1]
            tot2 = stats[:, 1:2] + rt[:, 1:2]
            mean = tot1 / N_GLOBAL
            var = tot2 / N_GLOBAL - mean * mean
            rstd = lax.rsqrt(var + EPS)
            stats[:, 0:1] = rstd
            stats[:, 1:2] = -mean * rstd

        @pl.when(j > 0)
        def norm_phase():
            sl = pl.ds((j - 1) * bm, bm)
            o_ref[...] = (
                (xv[sl, :] * stats[sl, 0:1] + stats[sl, 1:2]) * g_ref[...]
                + b_ref[...]
            ).astype(jnp.bfloat16)

    return pl.pallas_call(
        body,
        grid=(K + 1,),
        out_shape=jax.ShapeDtypeStruct((m, n), jnp.bfloat16),
        in_specs=[
            pl.BlockSpec(memory_space=pltpu.MemorySpace.HBM),
            pl.BlockSpec((1, n), lambda j: (0, 0)),
            pl.BlockSpec((1, n), lambda j: (0, 0)),
        ],
        out_specs=pl.BlockSpec((bm, n), lambda j: (jnp.maximum(j - 1, 0), 0)),
        scratch_shapes=[
            pltpu.VMEM((m, n), jnp.float32),
            pltpu.VMEM((m, 2), jnp.float32),
            pltpu.VMEM((2, m), jnp.float32),
            pltpu.VMEM((2, m), jnp.float32),
            pltpu.SemaphoreType.DMA((K,)),
            pltpu.SemaphoreType.DMA((2,)),
            pltpu.SemaphoreType.DMA((2,)),
        ],
        compiler_params=pltpu.CompilerParams(
            collective_id=0, vmem_limit_bytes=64 * 1024 * 1024
        ),
    )(x, gamma2, beta2)
